# baseline (speedup 1.0000x reference)
import jax
import jax.numpy as jnp
from jax import lax
from jax.experimental import pallas as pl
from jax.experimental.pallas import tpu as pltpu
from jax.experimental.pallas import tpu_sc as plsc

_W = 128
_SUPER = 16
_NW = 32
_LANES = 16
_NBUF = 4
_K = 5


def _slab_writer(batch, dim, seq_per, n_slabs, j, aliased):
    pack = 128 // dim
    bb = pack * _W
    nb = batch // bb

    def body(x_ref, *rest):
        y_ref = rest[-1]
        x = x_ref[...]
        y_ref[...] = jnp.concatenate(
            [x[:, u * dim : (u + 1) * dim].T for u in range(pack)], axis=1
        )

    out_shape = jax.ShapeDtypeStruct((n_slabs * seq_per * dim, batch), jnp.float32)
    in_specs = [
        pl.BlockSpec((_W, 128), lambda s, i: (s * nb + i, 0)),
    ]
    if aliased:
        in_specs.append(pl.BlockSpec(memory_space=pl.ANY))
    return pl.pallas_call(
        body,
        grid=(seq_per, nb),
        in_specs=in_specs,
        out_specs=pl.BlockSpec((dim, bb), lambda s, i: (j * seq_per + s, i)),
        out_shape=out_shape,
        input_output_aliases={1: 0} if aliased else {},
    )


def kernel(ids, weight):
    batch, seq = ids.shape
    n_all = batch * seq
    vocab, dim = weight.shape
    pack = 128 // dim
    all_ids = ids.T.reshape(n_all)
    w_packed = weight.reshape(vocab // pack, 128)

    n_ids = n_all // _K
    n_per_w = n_ids // _NW
    n_super = _SUPER * _W
    supers = n_per_w // n_super

    mesh = plsc.VectorSubcoreMesh(core_axis_name="core", subcore_axis_name="subcore")

    @pl.kernel(
        out_type=jax.ShapeDtypeStruct((n_ids * dim // 128, 128), weight.dtype),
        mesh=mesh,
        scratch_types=[
            pltpu.VMEM((n_super,), jnp.int32),
            pltpu.VMEM((n_super,), jnp.int32),
        ]
        + [pltpu.VMEM((_W, 128), jnp.float32)] * _NBUF
        + [pltpu.VMEM((_W, 128), jnp.float32)] * 2
        + [pltpu.SemaphoreType.DMA] * (_NBUF + 2),
    )
    def gather_kernel(w_hbm, i_hbm, o_hbm, idx_v, pidx_v, *bufs):
        rows = bufs[:_NBUF]
        outs = bufs[_NBUF : _NBUF + 2]
        gsems = bufs[_NBUF + 2 : 2 * _NBUF + 2]
        osems = bufs[2 * _NBUF + 2 :]
        wid = lax.axis_index("subcore") * 2 + lax.axis_index("core")
        base_w = wid * n_per_w

        def start_gather(c, slot):
            pltpu.async_copy(
                w_hbm.at[pidx_v.at[pl.ds(c * _W, _W)]], rows[slot], gsems[slot]
            )

        def wait_gather(c, slot):
            pltpu.make_async_copy(
                w_hbm.at[pidx_v.at[pl.ds(c * _W, _W)]], rows[slot], gsems[slot]
            ).wait()

        def extract(c, slot, gslot, u):
            @pl.loop(0, _W, step=_LANES)
            def _(b):
                offs = (idx_v[pl.ds(c * _W + b, _LANES)] & (pack - 1)) * dim
                for j in range(_LANES):
                    r = b + j
                    off = offs[j]
                    outs[gslot][r, pl.ds(u * dim, _LANES)] = rows[slot][
                        r, pl.ds(off, _LANES)
                    ]
                    outs[gslot][r, pl.ds(u * dim + _LANES, _LANES)] = rows[slot][
                        r, pl.ds(off + _LANES, _LANES)
                    ]

        @pl.loop(0, supers)
        def _(s):
            base = base_w + s * n_super
            pltpu.sync_copy(i_hbm.at[pl.ds(base, n_super)], idx_v)

            @pl.loop(0, n_super, step=_LANES)
            def _(j):
                pidx_v[pl.ds(j, _LANES)] = idx_v[pl.ds(j, _LANES)] >> 2

            for p in range(_NBUF - 1):
                start_gather(p, p)

            @pl.loop(0, _SUPER, step=2 * _NBUF)
            def _(c):
                for g2 in range(2):
                    gc = c + g2 * _NBUF

                    @pl.when((s > 0) | (gc >= 2 * _NBUF))
                    def _():
                        pltpu.make_async_copy(
                            outs[g2], o_hbm.at[pl.ds(0, _W)], osems[g2]
                        ).wait()

                    for u in range(_NBUF):
                        cc = gc + u

                        @pl.when(cc + _NBUF - 1 < _SUPER)
                        def _():
                            start_gather(cc + _NBUF - 1, (u + _NBUF - 1) % _NBUF)

                        wait_gather(cc, u)
                        extract(cc, u, g2, u)

                    row_off = pl.multiple_of((base + gc * _W) * dim // 128, 32)
                    pltpu.async_copy(
                        outs[g2],
                        o_hbm.at[pl.ds(row_off, _W)],
                        osems[g2],
                    )

        for g2 in range(2):
            pltpu.make_async_copy(
                outs[g2], o_hbm.at[pl.ds(0, _W)], osems[g2]
            ).wait()

    seq_per = seq // _K
    big = None
    for j in range(_K):
        part = gather_kernel(w_packed, all_ids[j * n_ids : (j + 1) * n_ids])
        writer = _slab_writer(batch, dim, seq_per, _K, j, aliased=j > 0)
        big = writer(part) if j == 0 else writer(part, big)
    return big.reshape(seq, dim, batch).transpose(2, 0, 1)


# speedup vs baseline: 1.0561x; 1.0561x over previous
import jax
import jax.numpy as jnp
from jax import lax
from jax.experimental import pallas as pl
from jax.experimental.pallas import tpu as pltpu
from jax.experimental.pallas import tpu_sc as plsc

_W = 128
_SUPER = 16
_NW = 32
_LANES = 16
_NBUF = 4
_K = 5


def _slab_writer(batch, dim, seq_per, n_slabs, j, aliased):
    pack = 128 // dim
    bb = pack * _W
    nb = batch // bb

    def body(x_ref, *rest):
        y_ref = rest[-1]
        xt = x_ref[...].T
        y_ref[...] = jnp.concatenate(
            [xt[u * dim : (u + 1) * dim, :] for u in range(pack)], axis=1
        )

    out_shape = jax.ShapeDtypeStruct((n_slabs * seq_per * dim, batch), jnp.float32)
    in_specs = [
        pl.BlockSpec((_W, 128), lambda s, i: (s * nb + i, 0)),
    ]
    if aliased:
        in_specs.append(pl.BlockSpec(memory_space=pl.ANY))
    return pl.pallas_call(
        body,
        grid=(seq_per, nb),
        in_specs=in_specs,
        out_specs=pl.BlockSpec((dim, bb), lambda s, i: (j * seq_per + s, i)),
        out_shape=out_shape,
        input_output_aliases={1: 0} if aliased else {},
    )


def kernel(ids, weight):
    batch, seq = ids.shape
    n_all = batch * seq
    vocab, dim = weight.shape
    pack = 128 // dim
    all_ids = ids.T.reshape(n_all)
    w_packed = weight.reshape(vocab // pack, 128)

    n_ids = n_all // _K
    n_per_w = n_ids // _NW
    n_super = _SUPER * _W
    supers = n_per_w // n_super

    mesh = plsc.VectorSubcoreMesh(core_axis_name="core", subcore_axis_name="subcore")

    @pl.kernel(
        out_type=jax.ShapeDtypeStruct((n_ids * dim // 128, 128), weight.dtype),
        mesh=mesh,
        scratch_types=[
            pltpu.VMEM((n_super,), jnp.int32),
            pltpu.VMEM((n_super,), jnp.int32),
        ]
        + [pltpu.VMEM((_W, 128), jnp.float32)] * _NBUF
        + [pltpu.VMEM((_W, 128), jnp.float32)] * 2
        + [pltpu.SemaphoreType.DMA] * (_NBUF + 2),
    )
    def gather_kernel(w_hbm, i_hbm, o_hbm, idx_v, pidx_v, *bufs):
        rows = bufs[:_NBUF]
        outs = bufs[_NBUF : _NBUF + 2]
        gsems = bufs[_NBUF + 2 : 2 * _NBUF + 2]
        osems = bufs[2 * _NBUF + 2 :]
        wid = lax.axis_index("subcore") * 2 + lax.axis_index("core")
        base_w = wid * n_per_w

        def start_gather(c, slot):
            pltpu.async_copy(
                w_hbm.at[pidx_v.at[pl.ds(c * _W, _W)]], rows[slot], gsems[slot]
            )

        def wait_gather(c, slot):
            pltpu.make_async_copy(
                w_hbm.at[pidx_v.at[pl.ds(c * _W, _W)]], rows[slot], gsems[slot]
            ).wait()

        def extract(c, slot, gslot, u):
            @pl.loop(0, _W, step=_LANES)
            def _(b):
                offs = (idx_v[pl.ds(c * _W + b, _LANES)] & (pack - 1)) * dim
                for j in range(_LANES):
                    r = b + j
                    off = offs[j]
                    outs[gslot][r, pl.ds(u * dim, _LANES)] = rows[slot][
                        r, pl.ds(off, _LANES)
                    ]
                    outs[gslot][r, pl.ds(u * dim + _LANES, _LANES)] = rows[slot][
                        r, pl.ds(off + _LANES, _LANES)
                    ]

        @pl.loop(0, supers)
        def _(s):
            base = base_w + s * n_super
            pltpu.sync_copy(i_hbm.at[pl.ds(base, n_super)], idx_v)

            @pl.loop(0, n_super, step=_LANES)
            def _(j):
                pidx_v[pl.ds(j, _LANES)] = idx_v[pl.ds(j, _LANES)] >> 2

            for p in range(_NBUF - 1):
                start_gather(p, p)

            @pl.loop(0, _SUPER, step=2 * _NBUF)
            def _(c):
                for g2 in range(2):
                    gc = c + g2 * _NBUF

                    @pl.when((s > 0) | (gc >= 2 * _NBUF))
                    def _():
                        pltpu.make_async_copy(
                            outs[g2], o_hbm.at[pl.ds(0, _W)], osems[g2]
                        ).wait()

                    for u in range(_NBUF):
                        cc = gc + u

                        @pl.when(cc + _NBUF - 1 < _SUPER)
                        def _():
                            start_gather(cc + _NBUF - 1, (u + _NBUF - 1) % _NBUF)

                        wait_gather(cc, u)
                        extract(cc, u, g2, u)

                    row_off = pl.multiple_of((base + gc * _W) * dim // 128, 32)
                    pltpu.async_copy(
                        outs[g2],
                        o_hbm.at[pl.ds(row_off, _W)],
                        osems[g2],
                    )

        for g2 in range(2):
            pltpu.make_async_copy(
                outs[g2], o_hbm.at[pl.ds(0, _W)], osems[g2]
            ).wait()

    seq_per = seq // _K
    big = None
    for j in range(_K):
        part = gather_kernel(w_packed, all_ids[j * n_ids : (j + 1) * n_ids])
        writer = _slab_writer(batch, dim, seq_per, _K, j, aliased=j > 0)
        big = writer(part) if j == 0 else writer(part, big)
    return big.reshape(seq, dim, batch).transpose(2, 0, 1)


# speedup vs baseline: 2.5832x; 2.4459x over previous
import jax
import jax.numpy as jnp
from jax import lax
from jax.experimental import pallas as pl
from jax.experimental.pallas import tpu as pltpu
from jax.experimental.pallas import tpu_sc as plsc

_W = 128
_SUPER = 16
_NW = 32
_LANES = 16
_NBUF = 4
_K = 5


def _slab_writer(batch, dim, seq_per, n_slabs, j, aliased):
    pack = 128 // dim
    bb = pack * _W
    gb = 8
    nb = batch // (bb * gb)

    def body(x_ref, *rest):
        y_ref = rest[-1]
        pieces = []
        for g in range(gb):
            xt = x_ref[g * _W : (g + 1) * _W, :].T
            pieces += [xt[u * dim : (u + 1) * dim, :] for u in range(pack)]
        y_ref[...] = jnp.concatenate(pieces, axis=1)

    out_shape = jax.ShapeDtypeStruct((n_slabs * seq_per * dim, batch), jnp.float32)
    in_specs = [
        pl.BlockSpec((_W * gb, 128), lambda s, i: (s * nb + i, 0)),
    ]
    if aliased:
        in_specs.append(pl.BlockSpec(memory_space=pl.ANY))
    return pl.pallas_call(
        body,
        grid=(seq_per, nb),
        in_specs=in_specs,
        out_specs=pl.BlockSpec((dim, bb * gb), lambda s, i: (j * seq_per + s, i)),
        out_shape=out_shape,
        input_output_aliases={1: 0} if aliased else {},
    )


def kernel(ids, weight):
    batch, seq = ids.shape
    n_all = batch * seq
    vocab, dim = weight.shape
    pack = 128 // dim
    all_ids = ids.T.reshape(n_all)
    w_packed = weight.reshape(vocab // pack, 128)

    n_ids = n_all // _K
    n_per_w = n_ids // _NW
    n_super = _SUPER * _W
    supers = n_per_w // n_super

    mesh = plsc.VectorSubcoreMesh(core_axis_name="core", subcore_axis_name="subcore")

    @pl.kernel(
        out_type=jax.ShapeDtypeStruct((n_ids * dim // 128, 128), weight.dtype),
        mesh=mesh,
        scratch_types=[
            pltpu.VMEM((n_super,), jnp.int32),
            pltpu.VMEM((n_super,), jnp.int32),
        ]
        + [pltpu.VMEM((_W, 128), jnp.float32)] * _NBUF
        + [pltpu.VMEM((_W, 128), jnp.float32)] * 2
        + [pltpu.SemaphoreType.DMA] * (_NBUF + 2),
    )
    def gather_kernel(w_hbm, i_hbm, o_hbm, idx_v, pidx_v, *bufs):
        rows = bufs[:_NBUF]
        outs = bufs[_NBUF : _NBUF + 2]
        gsems = bufs[_NBUF + 2 : 2 * _NBUF + 2]
        osems = bufs[2 * _NBUF + 2 :]
        wid = lax.axis_index("subcore") * 2 + lax.axis_index("core")
        base_w = wid * n_per_w

        def start_gather(c, slot):
            pltpu.async_copy(
                w_hbm.at[pidx_v.at[pl.ds(c * _W, _W)]], rows[slot], gsems[slot]
            )

        def wait_gather(c, slot):
            pltpu.make_async_copy(
                w_hbm.at[pidx_v.at[pl.ds(c * _W, _W)]], rows[slot], gsems[slot]
            ).wait()

        def extract(c, slot, gslot, u):
            @pl.loop(0, _W, step=_LANES)
            def _(b):
                offs = (idx_v[pl.ds(c * _W + b, _LANES)] & (pack - 1)) * dim
                for j in range(_LANES):
                    r = b + j
                    off = offs[j]
                    outs[gslot][r, pl.ds(u * dim, _LANES)] = rows[slot][
                        r, pl.ds(off, _LANES)
                    ]
                    outs[gslot][r, pl.ds(u * dim + _LANES, _LANES)] = rows[slot][
                        r, pl.ds(off + _LANES, _LANES)
                    ]

        @pl.loop(0, supers)
        def _(s):
            base = base_w + s * n_super
            pltpu.sync_copy(i_hbm.at[pl.ds(base, n_super)], idx_v)

            @pl.loop(0, n_super, step=_LANES)
            def _(j):
                pidx_v[pl.ds(j, _LANES)] = idx_v[pl.ds(j, _LANES)] >> 2

            for p in range(_NBUF - 1):
                start_gather(p, p)

            @pl.loop(0, _SUPER, step=2 * _NBUF)
            def _(c):
                for g2 in range(2):
                    gc = c + g2 * _NBUF

                    @pl.when((s > 0) | (gc >= 2 * _NBUF))
                    def _():
                        pltpu.make_async_copy(
                            outs[g2], o_hbm.at[pl.ds(0, _W)], osems[g2]
                        ).wait()

                    for u in range(_NBUF):
                        cc = gc + u

                        @pl.when(cc + _NBUF - 1 < _SUPER)
                        def _():
                            start_gather(cc + _NBUF - 1, (u + _NBUF - 1) % _NBUF)

                        wait_gather(cc, u)
                        extract(cc, u, g2, u)

                    row_off = pl.multiple_of((base + gc * _W) * dim // 128, 32)
                    pltpu.async_copy(
                        outs[g2],
                        o_hbm.at[pl.ds(row_off, _W)],
                        osems[g2],
                    )

        for g2 in range(2):
            pltpu.make_async_copy(
                outs[g2], o_hbm.at[pl.ds(0, _W)], osems[g2]
            ).wait()

    seq_per = seq // _K
    big = None
    for j in range(_K):
        part = gather_kernel(w_packed, all_ids[j * n_ids : (j + 1) * n_ids])
        writer = _slab_writer(batch, dim, seq_per, _K, j, aliased=j > 0)
        big = writer(part) if j == 0 else writer(part, big)
    return big.reshape(seq, dim, batch).transpose(2, 0, 1)


# speedup vs baseline: 2.5865x; 1.0013x over previous
import jax
import jax.numpy as jnp
from jax import lax
from jax.experimental import pallas as pl
from jax.experimental.pallas import tpu as pltpu
from jax.experimental.pallas import tpu_sc as plsc

_W = 128
_SUPER = 16
_NW = 32
_LANES = 16
_NBUF = 4
_K = 10


def _slab_writer(batch, dim, seq_per, n_slabs, j, aliased):
    pack = 128 // dim
    bb = pack * _W
    gb = 8
    nb = batch // (bb * gb)

    def body(x_ref, *rest):
        y_ref = rest[-1]
        pieces = []
        for g in range(gb):
            xt = x_ref[g * _W : (g + 1) * _W, :].T
            pieces += [xt[u * dim : (u + 1) * dim, :] for u in range(pack)]
        y_ref[...] = jnp.concatenate(pieces, axis=1)

    out_shape = jax.ShapeDtypeStruct((n_slabs * seq_per * dim, batch), jnp.float32)
    in_specs = [
        pl.BlockSpec((_W * gb, 128), lambda s, i: (s * nb + i, 0)),
    ]
    if aliased:
        in_specs.append(pl.BlockSpec(memory_space=pl.ANY))
    return pl.pallas_call(
        body,
        grid=(seq_per, nb),
        in_specs=in_specs,
        out_specs=pl.BlockSpec((dim, bb * gb), lambda s, i: (j * seq_per + s, i)),
        out_shape=out_shape,
        input_output_aliases={1: 0} if aliased else {},
    )


def kernel(ids, weight):
    batch, seq = ids.shape
    n_all = batch * seq
    vocab, dim = weight.shape
    pack = 128 // dim
    all_ids = ids.T.reshape(n_all)
    w_packed = weight.reshape(vocab // pack, 128)

    n_ids = n_all // _K
    n_per_w = n_ids // _NW
    n_super = _SUPER * _W
    supers = n_per_w // n_super

    mesh = plsc.VectorSubcoreMesh(core_axis_name="core", subcore_axis_name="subcore")

    @pl.kernel(
        out_type=jax.ShapeDtypeStruct((n_ids * dim // 128, 128), weight.dtype),
        mesh=mesh,
        scratch_types=[
            pltpu.VMEM((n_super,), jnp.int32),
            pltpu.VMEM((n_super,), jnp.int32),
        ]
        + [pltpu.VMEM((_W, 128), jnp.float32)] * _NBUF
        + [pltpu.VMEM((_W, 128), jnp.float32)] * 2
        + [pltpu.SemaphoreType.DMA] * (_NBUF + 2),
    )
    def gather_kernel(w_hbm, i_hbm, o_hbm, idx_v, pidx_v, *bufs):
        rows = bufs[:_NBUF]
        outs = bufs[_NBUF : _NBUF + 2]
        gsems = bufs[_NBUF + 2 : 2 * _NBUF + 2]
        osems = bufs[2 * _NBUF + 2 :]
        wid = lax.axis_index("subcore") * 2 + lax.axis_index("core")
        base_w = wid * n_per_w

        def start_gather(c, slot):
            pltpu.async_copy(
                w_hbm.at[pidx_v.at[pl.ds(c * _W, _W)]], rows[slot], gsems[slot]
            )

        def wait_gather(c, slot):
            pltpu.make_async_copy(
                w_hbm.at[pidx_v.at[pl.ds(c * _W, _W)]], rows[slot], gsems[slot]
            ).wait()

        def extract(c, slot, gslot, u):
            @pl.loop(0, _W, step=_LANES)
            def _(b):
                offs = (idx_v[pl.ds(c * _W + b, _LANES)] & (pack - 1)) * dim
                for j in range(_LANES):
                    r = b + j
                    off = offs[j]
                    outs[gslot][r, pl.ds(u * dim, _LANES)] = rows[slot][
                        r, pl.ds(off, _LANES)
                    ]
                    outs[gslot][r, pl.ds(u * dim + _LANES, _LANES)] = rows[slot][
                        r, pl.ds(off + _LANES, _LANES)
                    ]

        @pl.loop(0, supers)
        def _(s):
            base = base_w + s * n_super
            pltpu.sync_copy(i_hbm.at[pl.ds(base, n_super)], idx_v)

            @pl.loop(0, n_super, step=_LANES)
            def _(j):
                pidx_v[pl.ds(j, _LANES)] = idx_v[pl.ds(j, _LANES)] >> 2

            for p in range(_NBUF - 1):
                start_gather(p, p)

            @pl.loop(0, _SUPER, step=2 * _NBUF)
            def _(c):
                for g2 in range(2):
                    gc = c + g2 * _NBUF

                    @pl.when((s > 0) | (gc >= 2 * _NBUF))
                    def _():
                        pltpu.make_async_copy(
                            outs[g2], o_hbm.at[pl.ds(0, _W)], osems[g2]
                        ).wait()

                    for u in range(_NBUF):
                        cc = gc + u

                        @pl.when(cc + _NBUF - 1 < _SUPER)
                        def _():
                            start_gather(cc + _NBUF - 1, (u + _NBUF - 1) % _NBUF)

                        wait_gather(cc, u)
                        extract(cc, u, g2, u)

                    row_off = pl.multiple_of((base + gc * _W) * dim // 128, 32)
                    pltpu.async_copy(
                        outs[g2],
                        o_hbm.at[pl.ds(row_off, _W)],
                        osems[g2],
                    )

        for g2 in range(2):
            pltpu.make_async_copy(
                outs[g2], o_hbm.at[pl.ds(0, _W)], osems[g2]
            ).wait()

    seq_per = seq // _K
    big = None
    for j in range(_K):
        part = gather_kernel(w_packed, all_ids[j * n_ids : (j + 1) * n_ids])
        writer = _slab_writer(batch, dim, seq_per, _K, j, aliased=j > 0)
        big = writer(part) if j == 0 else writer(part, big)
    return big.reshape(seq, dim, batch).transpose(2, 0, 1)
